# Initial kernel scaffold; baseline (speedup 1.0000x reference)
import jax
import jax.numpy as jnp
from jax.experimental import pallas as pl


def kernel(coke_features, keypoint_positions, has_smpl, iskpvisible, feature_bank, adj_mat, vert_orients, bg_mask):
    raise NotImplementedError("write your pallas kernel here")


# speedup vs baseline: 1.2291x; 1.2291x over previous
import functools
import numpy as np
import jax
import jax.numpy as jnp
from jax.experimental import pallas as pl
from jax.experimental.pallas import tpu as pltpu

N_NOISE = 64
NUM_NEG = 128
N_ORIENT = 3
T = 0.07
WEIGHT_NOISE = 0.005
EPS = 100000.0
NEG_CONST = float(-np.log(WEIGHT_NOISE))


def _loss_body(xg_ref, bankT_ref, adjf_ref, vo_ref, vis_ref, o1_ref, o2_ref, o3_ref):
    xg = xg_ref[0]
    ssq = jnp.sum(xg * xg, axis=1, keepdims=True)
    inv = 1.0 / jnp.maximum(jnp.sqrt(ssq), 1e-12)
    feats = xg * inv
    sim = jnp.dot(feats, bankT_ref[...], preferred_element_type=jnp.float32)
    invT = 1.0 / T
    simk = sim[:512, :]
    simn = sim[512:, :]

    rows = jax.lax.broadcasted_iota(jnp.int32, (512, 512), 0)
    cols = jax.lax.broadcasted_iota(jnp.int32, (512, 512), 1)
    noneye = (rows != cols).astype(jnp.float32)
    eye = 1.0 - noneye
    adjm = EPS * (adjf_ref[...] * noneye)

    l0 = simk[:, 0:512] * invT - adjm
    l1 = simk[:, 512:1024] * invT - adjm
    l2 = simk[:, 1024:1536] * invT - adjm
    l3 = simk[:, 1536:1664] * invT - NEG_CONST

    m01 = jnp.maximum(jnp.max(l0, axis=1), jnp.max(l1, axis=1))
    m23 = jnp.maximum(jnp.max(l2, axis=1), jnp.max(l3, axis=1))
    m = jnp.maximum(m01, m23)[:, None]
    se = (jnp.sum(jnp.exp(l0 - m), axis=1) + jnp.sum(jnp.exp(l1 - m), axis=1)
          + jnp.sum(jnp.exp(l2 - m), axis=1) + jnp.sum(jnp.exp(l3 - m), axis=1))
    lse = m[:, 0] + jnp.log(se)

    d0 = jnp.sum(l0 * eye, axis=1)
    d1 = jnp.sum(l1 * eye, axis=1)
    d2 = jnp.sum(l2 * eye, axis=1)
    vo = vo_ref[0, 0]
    dlab = jnp.where(vo == 0, d0, jnp.where(vo == 1, d1, d2))
    nll = lse - dlab
    vis = vis_ref[0, 0]

    snll = jnp.sum(nll * vis)
    svis = jnp.sum(vis)
    snoise = jnp.sum(simn[:, :1536]) * invT

    o1_ref[...] = jnp.full((1, 1, 128), snll, dtype=jnp.float32)
    o2_ref[...] = jnp.full((1, 1, 128), svis, dtype=jnp.float32)
    o3_ref[...] = jnp.full((1, 1, 128), snoise, dtype=jnp.float32)


def kernel(coke_features, keypoint_positions, has_smpl, iskpvisible, feature_bank,
           adj_mat, vert_orients, bg_mask):
    n, c, h, w = coke_features.shape
    k = keypoint_positions.shape[1]
    hw = h * w
    r = keypoint_positions[:, :, 0]
    col = keypoint_positions[:, :, 1]
    invisible = (r < 0) | (col < 0) | (r > h - 1) | (col > w - 1)
    vis = iskpvisible.astype(jnp.float32) * (~invisible).astype(jnp.float32)
    r = jnp.where(invisible, 0, r)
    col = jnp.where(invisible, 0, col)
    vis = jnp.where((has_smpl == 0)[:, None], 0.0, vis)
    kp_idx = r * w + col

    mask = jnp.ones((n, hw), dtype=jnp.float32)
    bidx = jnp.arange(n)[:, None]
    mask = mask.at[bidx, kp_idx].set(0.0)
    mask = mask * bg_mask.reshape(n, -1)
    logw = jnp.where(mask > 0, jnp.log(jnp.maximum(mask, 1e-12)), -1e9)
    u = jax.random.uniform(jax.random.key(1), (n, hw), minval=1e-6, maxval=1.0 - 1e-6)
    g = -jnp.log(-jnp.log(u))
    _, noise_idx = jax.lax.top_k(logw + g, N_NOISE)
    all_idx = jnp.concatenate([kp_idx, noise_idx], axis=1)

    X = coke_features.reshape(n, c, hw).transpose(0, 2, 1)
    idx_full = jnp.broadcast_to(all_idx[:, :, None], (n, all_idx.shape[1], c))
    xg = jnp.take_along_axis(X, idx_full, axis=1)

    bankT = feature_bank.T
    adjf = adj_mat[0].astype(jnp.float32)
    vo3 = vert_orients.astype(jnp.int32).reshape(n, 1, k)
    vis3 = vis.reshape(n, 1, k)

    grid = (n,)
    o1, o2, o3 = pl.pallas_call(
        _loss_body,
        grid=grid,
        in_specs=[
            pl.BlockSpec((1, k + N_NOISE, c), lambda i: (i, 0, 0)),
            pl.BlockSpec((c, 1664), lambda i: (0, 0)),
            pl.BlockSpec((k, k), lambda i: (0, 0)),
            pl.BlockSpec((1, 1, k), lambda i: (i, 0, 0)),
            pl.BlockSpec((1, 1, k), lambda i: (i, 0, 0)),
        ],
        out_specs=[
            pl.BlockSpec((1, 1, 128), lambda i: (i, 0, 0)),
            pl.BlockSpec((1, 1, 128), lambda i: (i, 0, 0)),
            pl.BlockSpec((1, 1, 128), lambda i: (i, 0, 0)),
        ],
        out_shape=[
            jax.ShapeDtypeStruct((n, 1, 128), jnp.float32),
            jax.ShapeDtypeStruct((n, 1, 128), jnp.float32),
            jax.ShapeDtypeStruct((n, 1, 128), jnp.float32),
        ],
    )(xg, bankT, adjf, vo3, vis3)

    snll = jnp.sum(o1[:, 0, 0])
    svis = jnp.sum(o2[:, 0, 0])
    snoise = jnp.sum(o3[:, 0, 0])
    loss_c = snll / jnp.maximum(svis, 1.0)
    loss_n = snoise / (n * N_NOISE * N_ORIENT * k)
    return loss_c + loss_n


# speedup vs baseline: 1.2304x; 1.0010x over previous
import functools
import numpy as np
import jax
import jax.numpy as jnp
from jax import lax
from jax.experimental import pallas as pl
from jax.experimental.pallas import tpu as pltpu

N_NOISE = 64
NUM_NEG = 128
N_ORIENT = 3
T = 0.07
WEIGHT_NOISE = 0.005
EPS = 100000.0
NEG_CONST = float(-np.log(WEIGHT_NOISE))


def _loss_body(xg_ref, bankT_ref, adjf_ref, vo_ref, vis_ref, o1_ref, o2_ref, o3_ref):
    xg = xg_ref[0]
    ssq = jnp.sum(xg * xg, axis=1, keepdims=True)
    inv = 1.0 / jnp.maximum(jnp.sqrt(ssq), 1e-12)
    feats = xg * inv
    sim = jnp.dot(feats, bankT_ref[...], preferred_element_type=jnp.float32)
    invT = 1.0 / T
    simk = sim[:512, :]
    simn = sim[512:, :]

    rows = jax.lax.broadcasted_iota(jnp.int32, (512, 512), 0)
    cols = jax.lax.broadcasted_iota(jnp.int32, (512, 512), 1)
    noneye = (rows != cols).astype(jnp.float32)
    eye = 1.0 - noneye
    adjm = EPS * (adjf_ref[...] * noneye)

    l0 = simk[:, 0:512] * invT - adjm
    l1 = simk[:, 512:1024] * invT - adjm
    l2 = simk[:, 1024:1536] * invT - adjm
    l3 = simk[:, 1536:1664] * invT - NEG_CONST

    m01 = jnp.maximum(jnp.max(l0, axis=1), jnp.max(l1, axis=1))
    m23 = jnp.maximum(jnp.max(l2, axis=1), jnp.max(l3, axis=1))
    m = jnp.maximum(m01, m23)[:, None]
    se = (jnp.sum(jnp.exp(l0 - m), axis=1) + jnp.sum(jnp.exp(l1 - m), axis=1)
          + jnp.sum(jnp.exp(l2 - m), axis=1) + jnp.sum(jnp.exp(l3 - m), axis=1))
    lse = m[:, 0] + jnp.log(se)

    d0 = jnp.sum(l0 * eye, axis=1)
    d1 = jnp.sum(l1 * eye, axis=1)
    d2 = jnp.sum(l2 * eye, axis=1)
    vo = vo_ref[0, 0]
    dlab = jnp.where(vo == 0, d0, jnp.where(vo == 1, d1, d2))
    nll = lse - dlab
    vis = vis_ref[0, 0]

    snll = jnp.sum(nll * vis)
    svis = jnp.sum(vis)
    snoise = jnp.sum(simn[:, :1536]) * invT

    o1_ref[...] = jnp.full((1, 1, 128), snll, dtype=jnp.float32)
    o2_ref[...] = jnp.full((1, 1, 128), svis, dtype=jnp.float32)
    o3_ref[...] = jnp.full((1, 1, 128), snoise, dtype=jnp.float32)


def kernel(coke_features, keypoint_positions, has_smpl, iskpvisible, feature_bank,
           adj_mat, vert_orients, bg_mask):
    n, c, h, w = coke_features.shape
    k = keypoint_positions.shape[1]
    hw = h * w
    r = keypoint_positions[:, :, 0]
    col = keypoint_positions[:, :, 1]
    invisible = (r < 0) | (col < 0) | (r > h - 1) | (col > w - 1)
    vis = iskpvisible.astype(jnp.float32) * (~invisible).astype(jnp.float32)
    r = jnp.where(invisible, 0, r)
    col = jnp.where(invisible, 0, col)
    vis = jnp.where((has_smpl == 0)[:, None], 0.0, vis)
    kp_idx = r * w + col

    mask = jnp.ones((n, hw), dtype=jnp.float32)
    bidx = jnp.arange(n)[:, None]
    mask = mask.at[bidx, kp_idx].set(0.0)
    mask = mask * bg_mask.reshape(n, -1)
    logw = jnp.where(mask > 0, jnp.log(jnp.maximum(mask, 1e-12)), -1e9)
    u = jax.random.uniform(jax.random.key(1), (n, hw), minval=1e-6, maxval=1.0 - 1e-6)
    g = -jnp.log(-jnp.log(u))
    _, noise_idx = jax.lax.top_k(logw + g, N_NOISE)
    all_idx = jnp.concatenate([kp_idx, noise_idx], axis=1)

    X = coke_features.reshape(n, c, hw).transpose(0, 2, 1)
    idx_full = jnp.broadcast_to(all_idx[:, :, None], (n, all_idx.shape[1], c))
    xg = jnp.take_along_axis(X, idx_full, axis=1)

    bankT = feature_bank.T
    adjf = adj_mat[0].astype(jnp.float32)
    vo3 = vert_orients.astype(jnp.int32).reshape(n, 1, k)
    vis3 = vis.reshape(n, 1, k)

    grid = (n,)
    o1, o2, o3 = pl.pallas_call(
        _loss_body,
        grid=grid,
        in_specs=[
            pl.BlockSpec((1, k + N_NOISE, c), lambda i: (i, 0, 0)),
            pl.BlockSpec((c, 1664), lambda i: (0, 0)),
            pl.BlockSpec((k, k), lambda i: (0, 0)),
            pl.BlockSpec((1, 1, k), lambda i: (i, 0, 0)),
            pl.BlockSpec((1, 1, k), lambda i: (i, 0, 0)),
        ],
        out_specs=[
            pl.BlockSpec((1, 1, 128), lambda i: (i, 0, 0)),
            pl.BlockSpec((1, 1, 128), lambda i: (i, 0, 0)),
            pl.BlockSpec((1, 1, 128), lambda i: (i, 0, 0)),
        ],
        out_shape=[
            jax.ShapeDtypeStruct((n, 1, 128), jnp.float32),
            jax.ShapeDtypeStruct((n, 1, 128), jnp.float32),
            jax.ShapeDtypeStruct((n, 1, 128), jnp.float32),
        ],
    )(xg, bankT, adjf, vo3, vis3)

    snll = jnp.sum(o1[:, 0, 0])
    svis = jnp.sum(o2[:, 0, 0])
    snoise = jnp.sum(o3[:, 0, 0])
    loss_c = snll / jnp.maximum(svis, 1.0)
    loss_n = snoise / (n * N_NOISE * N_ORIENT * k)
    return loss_c + loss_n


# speedup vs baseline: 1.3081x; 1.0632x over previous
import functools
import numpy as np
import jax
import jax.numpy as jnp
from jax import lax
from jax.experimental import pallas as pl
from jax.experimental.pallas import tpu as pltpu

N_NOISE = 64
NUM_NEG = 128
N_ORIENT = 3
T = 0.07
WEIGHT_NOISE = 0.005
EPS = 100000.0
NEG_CONST = float(-np.log(WEIGHT_NOISE))


def _loss_body(xg_ref, bankT_ref, adjf_ref, vo_ref, vis_ref, o1_ref, o2_ref, o3_ref):
    xg = xg_ref[0]
    ssq = jnp.sum(xg * xg, axis=1, keepdims=True)
    inv = 1.0 / jnp.maximum(jnp.sqrt(ssq), 1e-12)
    feats = xg * inv
    sim = jnp.dot(feats, bankT_ref[...], preferred_element_type=jnp.float32)
    invT = 1.0 / T
    simk = sim[:512, :]
    simn = sim[512:, :]

    rows = jax.lax.broadcasted_iota(jnp.int32, (512, 512), 0)
    cols = jax.lax.broadcasted_iota(jnp.int32, (512, 512), 1)
    noneye = (rows != cols).astype(jnp.float32)
    eye = 1.0 - noneye
    adjm = EPS * (adjf_ref[...] * noneye)

    l0 = simk[:, 0:512] * invT - adjm
    l1 = simk[:, 512:1024] * invT - adjm
    l2 = simk[:, 1024:1536] * invT - adjm
    l3 = simk[:, 1536:1664] * invT - NEG_CONST

    m01 = jnp.maximum(jnp.max(l0, axis=1), jnp.max(l1, axis=1))
    m23 = jnp.maximum(jnp.max(l2, axis=1), jnp.max(l3, axis=1))
    m = jnp.maximum(m01, m23)[:, None]
    se = (jnp.sum(jnp.exp(l0 - m), axis=1) + jnp.sum(jnp.exp(l1 - m), axis=1)
          + jnp.sum(jnp.exp(l2 - m), axis=1) + jnp.sum(jnp.exp(l3 - m), axis=1))
    lse = m[:, 0] + jnp.log(se)

    d0 = jnp.sum(l0 * eye, axis=1)
    d1 = jnp.sum(l1 * eye, axis=1)
    d2 = jnp.sum(l2 * eye, axis=1)
    vo = vo_ref[0, 0]
    dlab = jnp.where(vo == 0, d0, jnp.where(vo == 1, d1, d2))
    nll = lse - dlab
    vis = vis_ref[0, 0]

    snll = jnp.sum(nll * vis)
    svis = jnp.sum(vis)
    snoise = jnp.sum(simn[:, :1536]) * invT

    o1_ref[...] = jnp.full((1, 1, 128), snll, dtype=jnp.float32)
    o2_ref[...] = jnp.full((1, 1, 128), svis, dtype=jnp.float32)
    o3_ref[...] = jnp.full((1, 1, 128), snoise, dtype=jnp.float32)


def kernel(coke_features, keypoint_positions, has_smpl, iskpvisible, feature_bank,
           adj_mat, vert_orients, bg_mask):
    n, c, h, w = coke_features.shape
    k = keypoint_positions.shape[1]
    hw = h * w
    r = keypoint_positions[:, :, 0]
    col = keypoint_positions[:, :, 1]
    invisible = (r < 0) | (col < 0) | (r > h - 1) | (col > w - 1)
    vis = iskpvisible.astype(jnp.float32) * (~invisible).astype(jnp.float32)
    r = jnp.where(invisible, 0, r)
    col = jnp.where(invisible, 0, col)
    vis = jnp.where((has_smpl == 0)[:, None], 0.0, vis)
    kp_idx = r * w + col

    onehot_r = (jnp.arange(h)[None, :, None] == r[:, None, :]).astype(jnp.float32)
    onehot_c = (jnp.arange(w)[None, None, :] == col[:, :, None]).astype(jnp.float32)
    hits = jnp.einsum('nrk,nkc->nrc', onehot_r, onehot_c)
    mask = jnp.where(hits.reshape(n, hw) == 0, bg_mask.reshape(n, -1), 0.0)
    logw = jnp.where(mask > 0, jnp.log(jnp.maximum(mask, 1e-12)), -1e9)
    u = jax.random.uniform(jax.random.key(1), (n, hw), minval=1e-6, maxval=1.0 - 1e-6)
    g = -jnp.log(-jnp.log(u))
    _, noise_idx = jax.lax.top_k(logw + g, N_NOISE)
    all_idx = jnp.concatenate([kp_idx, noise_idx], axis=1)

    X = coke_features.reshape(n, c, hw).transpose(0, 2, 1)
    idx_full = jnp.broadcast_to(all_idx[:, :, None], (n, all_idx.shape[1], c))
    xg = jnp.take_along_axis(X, idx_full, axis=1)

    bankT = feature_bank.T
    adjf = adj_mat[0].astype(jnp.float32)
    vo3 = vert_orients.astype(jnp.int32).reshape(n, 1, k)
    vis3 = vis.reshape(n, 1, k)

    grid = (n,)
    o1, o2, o3 = pl.pallas_call(
        _loss_body,
        grid=grid,
        in_specs=[
            pl.BlockSpec((1, k + N_NOISE, c), lambda i: (i, 0, 0)),
            pl.BlockSpec((c, 1664), lambda i: (0, 0)),
            pl.BlockSpec((k, k), lambda i: (0, 0)),
            pl.BlockSpec((1, 1, k), lambda i: (i, 0, 0)),
            pl.BlockSpec((1, 1, k), lambda i: (i, 0, 0)),
        ],
        out_specs=[
            pl.BlockSpec((1, 1, 128), lambda i: (i, 0, 0)),
            pl.BlockSpec((1, 1, 128), lambda i: (i, 0, 0)),
            pl.BlockSpec((1, 1, 128), lambda i: (i, 0, 0)),
        ],
        out_shape=[
            jax.ShapeDtypeStruct((n, 1, 128), jnp.float32),
            jax.ShapeDtypeStruct((n, 1, 128), jnp.float32),
            jax.ShapeDtypeStruct((n, 1, 128), jnp.float32),
        ],
    )(xg, bankT, adjf, vo3, vis3)

    snll = jnp.sum(o1[:, 0, 0])
    svis = jnp.sum(o2[:, 0, 0])
    snoise = jnp.sum(o3[:, 0, 0])
    loss_c = snll / jnp.maximum(svis, 1.0)
    loss_n = snoise / (n * N_NOISE * N_ORIENT * k)
    return loss_c + loss_n
